# baseline (speedup 1.0000x reference)
import jax
import jax.numpy as jnp
from jax import lax
from jax.experimental import pallas as pl
from jax.experimental.pallas import tpu as pltpu
from jax.experimental.pallas import tpu_sc as plsc

VOCAB = 1000000
EMB = 32
BATCH = 16384
HIST = 200

NC = 2
NS = 16
NW = NC * NS
NBLK = BATCH // 128
BLK_PER_W = NBLK // NW


def _emb_body(table_hbm, seqt_hbm, out_hbm, idx_v, rows_v, ptile_v,
              sem_g, sem_out, sem_idx):
    wid = lax.axis_index("s") * NC + lax.axis_index("c")
    c_iota = lax.iota(jnp.int32, 16)
    c3a = c_iota >> 3
    c7a = c_iota & 7
    c3b = c3a + 2
    zv = c_iota & 0

    ND = 4

    def gather(t, d):
        return pltpu.async_copy(table_hbm.at[idx_v.at[t]], rows_v.at[d],
                                sem_g.at[d])

    for a in range(BLK_PER_W):
        blk = wid * BLK_PER_W + a
        b0 = blk * 128

        pltpu.make_async_copy(
            seqt_hbm.at[:, pl.ds(b0, 128)], idx_v, sem_idx).start()
        pltpu.make_async_copy(
            seqt_hbm.at[:, pl.ds(b0, 128)], idx_v, sem_idx).wait()

        for d0 in range(4):
            gather(d0, d0)

        def outer(t2, carry, a=a, blk=blk):
            for d in range(4):
                t = t2 * 4 + d

                def out_copy(tt, d=d, blk=blk):
                    return pltpu.make_async_copy(
                        ptile_v.at[d],
                        out_hbm.at[tt, :, pl.ds(blk, 1)],
                        sem_out.at[d])

                pltpu.make_async_copy(
                    table_hbm.at[idx_v.at[t]], rows_v.at[d],
                    sem_g.at[d]).wait()

                if a == 0:
                    @pl.when(t >= 4)
                    def _():
                        out_copy(t).wait()
                else:
                    out_copy(t).wait()

                def transpose(j, carry2, d=d):
                    r0 = rows_v[d, j, pl.ds(0, 16)]
                    r1 = rows_v[d, j, pl.ds(16, 16)]
                    jv = zv + j
                    plsc.store_scatter(ptile_v.at[d], [c3a, zv, c7a, jv], r0)
                    plsc.store_scatter(ptile_v.at[d], [c3b, zv, c7a, jv], r1)
                    return carry2

                lax.fori_loop(0, 128, transpose, 0, unroll=16)

                @pl.when(t + 4 < HIST)
                def _():
                    gather(t + 4, d)

                out_copy(t).start()
            return carry

        lax.fori_loop(0, HIST // 4, outer, 0)
    for d in range(4):
        pltpu.make_async_copy(
            ptile_v.at[d], out_hbm.at[0, :, pl.ds(0, 1)],
            sem_out.at[d]).wait()


def kernel(seq, W):
    seqt = seq.T
    mesh = plsc.VectorSubcoreMesh(core_axis_name="c", subcore_axis_name="s")
    f = pl.kernel(
        _emb_body,
        out_type=jax.ShapeDtypeStruct((HIST, 4, NBLK, 8, 128), jnp.float32),
        mesh=mesh,
        scratch_types=[
            pltpu.VMEM((HIST, 128), jnp.int32),
            pltpu.VMEM((4, 128, EMB), jnp.float32),
            pltpu.VMEM((4, 4, 1, 8, 128), jnp.float32),
            pltpu.SemaphoreType.DMA((4,)),
            pltpu.SemaphoreType.DMA((4,)),
            pltpu.SemaphoreType.DMA,
        ],
        compiler_params=pltpu.CompilerParams(
            use_tc_tiling_on_sc=False, needs_layout_passes=False),
    )
    p5 = f(W, seqt)
    return p5.transpose(2, 4, 0, 1, 3).reshape(BATCH, HIST, EMB)


# speedup vs baseline: 1.0030x; 1.0030x over previous
import jax
import jax.numpy as jnp
from jax import lax
from jax.experimental import pallas as pl
from jax.experimental.pallas import tpu as pltpu
from jax.experimental.pallas import tpu_sc as plsc

VOCAB = 1000000
EMB = 32
BATCH = 16384
HIST = 200

B = BATCH * HIST
NC = 2
NS = 16
NW = NC * NS
PER_W = B // NW
CHUNK = 1024
NSLAB = PER_W // CHUNK
NB = 2


def _emb_body(table_hbm, idx_hbm, out_hbm, idx_v, rows_v, sem_idx, sem_g,
              sem_out):
    wid = lax.axis_index("s") * NC + lax.axis_index("c")
    base = wid * PER_W

    def idx_copy(s, b):
        return pltpu.make_async_copy(
            idx_hbm.at[pl.ds(base + s * CHUNK, CHUNK)], idx_v.at[b],
            sem_idx.at[b])

    def out_copy(s, b):
        return pltpu.make_async_copy(
            rows_v.at[b], out_hbm.at[pl.ds(base + s * CHUNK, CHUNK)],
            sem_out.at[b])

    idx_copy(0, 0).start()

    def outer(g, carry):
        for b in range(NB):
            s = g * NB + b
            idx_copy(s, b).wait()

            @pl.when(s + 1 < NSLAB)
            def _():
                idx_copy(s + 1, (b + 1) % NB).start()

            @pl.when(s >= NB)
            def _():
                out_copy(s, b).wait()

            pltpu.async_copy(table_hbm.at[idx_v.at[b]], rows_v.at[b],
                             sem_g).wait()
            out_copy(s, b).start()
        return carry

    lax.fori_loop(0, NSLAB // NB, outer, 0)
    for b in range(NB):
        out_copy(b, b).wait()


def kernel(seq, W):
    idx = seq.reshape(B).astype(jnp.int32)
    mesh = plsc.VectorSubcoreMesh(core_axis_name="c", subcore_axis_name="s")
    f = pl.kernel(
        _emb_body,
        out_type=jax.ShapeDtypeStruct((B, EMB), jnp.float32),
        mesh=mesh,
        scratch_types=[
            pltpu.VMEM((NB, CHUNK), jnp.int32),
            pltpu.VMEM((NB, CHUNK, EMB), jnp.float32),
            pltpu.SemaphoreType.DMA((NB,)),
            pltpu.SemaphoreType.DMA,
            pltpu.SemaphoreType.DMA((NB,)),
        ],
        compiler_params=pltpu.CompilerParams(use_tc_tiling_on_sc=False),
    )
    out = f(W, idx)
    return out.reshape(BATCH, HIST, EMB)
